# Initial kernel scaffold; baseline (speedup 1.0000x reference)
import functools

import jax
import jax.numpy as jnp
from jax import lax
from jax.experimental import pallas as pl
from jax.experimental.pallas import tpu as pltpu
from jax.experimental.pallas import tpu_sc as plsc

NC = 2
NS = 16
L = 16


@functools.lru_cache(maxsize=None)
def _build(M, D, B):
    NCH = 8
    MC = -(-M // NCH)
    MC = -(-MC // NS) * NS
    RPT = MC // NS
    BPT = B // NS
    assert BPT * NS == B and BPT % L == 0
    assert M - MC >= 0

    mesh = plsc.VectorSubcoreMesh(
        core_axis_name="c", subcore_axis_name="s",
        num_cores=NC, num_subcores=NS)

    @functools.partial(
        pl.kernel,
        out_type=jax.ShapeDtypeStruct((M, D), jnp.float32),
        mesh=mesh,
        scratch_types=[
            pltpu.VMEM_SHARED((MC + NS, D), jnp.float32),
            pltpu.VMEM((BPT,), jnp.int32),
            pltpu.VMEM((BPT + L,), jnp.int32),
            pltpu.VMEM((BPT + L,), jnp.int32),
            pltpu.VMEM((L, D), jnp.float32),
            pltpu.VMEM((L,), jnp.float32),
        ],
    )
    def _ker(x_hbm, idx_hbm, src_hbm, alpha_hbm, out_hbm,
             acc, idx_v, selb_v, selr_v, gsrc_v, alpha_v):
        c = lax.axis_index("c")
        s = lax.axis_index("s")
        trash = MC + s

        pltpu.sync_copy(idx_hbm.at[pl.ds(s * BPT, BPT)], idx_v)
        pltpu.sync_copy(alpha_hbm, alpha_v)
        av = alpha_v[...]

        def chunk_body(i, _):
            chunk = NCH // NC * c + i
            lo = jnp.minimum(chunk * MC, M - MC)
            hi = lo + MC
            base = lo + s * RPT

            pltpu.sync_copy(x_hbm.at[pl.ds(base, RPT)],
                            acc.at[pl.ds(s * RPT, RPT)])
            plsc.subcore_barrier()

            def sel_body(j, cnt):
                v = idx_v[pl.ds(j * L, L)]
                m = (v >= lo) & (v < hi)
                bsrc = s * BPT + j * L + lax.iota(jnp.int32, L)
                plsc.store_compressed(selb_v.at[pl.ds(cnt, L)], bsrc, mask=m)
                plsc.store_compressed(selr_v.at[pl.ds(cnt, L)], v - lo, mask=m)
                return cnt + jnp.sum(m.astype(jnp.int32))

            cnt = lax.fori_loop(0, BPT // L, sel_body, jnp.int32(0))

            selb_v[pl.ds(cnt, L)] = jnp.zeros((L,), jnp.int32)
            selr_v[pl.ds(cnt, L)] = jnp.broadcast_to(trash, (L,)).astype(jnp.int32)
            nb = (cnt + L - 1) // L

            def proc_body(j, _):
                bv = selb_v[pl.ds(j * L, L)]
                rv = selr_v[pl.ds(j * L, L)]
                pltpu.sync_copy(src_hbm.at[bv], gsrc_v)
                for r in range(L):
                    for cb in range(D // L):
                        sl = pl.ds(cb * L, L)
                        gsrc_v[r, sl] = gsrc_v[r, sl] * av
                pltpu.sync_copy(gsrc_v, acc.at[rv], add=True)
                return 0

            lax.fori_loop(0, nb, proc_body, 0)
            plsc.subcore_barrier()

            pltpu.sync_copy(acc.at[pl.ds(s * RPT, RPT)],
                            out_hbm.at[pl.ds(base, RPT)])
            plsc.subcore_barrier()
            return 0

        lax.fori_loop(0, NCH // NC, chunk_body, 0)

    return _ker


def kernel(x, dim, index, source, alpha, out):
    M, D = x.shape
    B = index.shape[0]
    alpha_arr = jnp.full((L,), alpha, jnp.float32)
    return _build(M, D, B)(x, index.astype(jnp.int32), source, alpha_arr)


# speedup vs baseline: 1.1658x; 1.1658x over previous
import functools

import jax
import jax.numpy as jnp
from jax import lax
from jax.experimental import pallas as pl
from jax.experimental.pallas import tpu as pltpu
from jax.experimental.pallas import tpu_sc as plsc

NC = 2
NS = 16
L = 16


@functools.lru_cache(maxsize=None)
def _build(M, D, B):
    NCH = 8
    MC = -(-M // NCH)
    MC = -(-MC // (NS * 8)) * (NS * 8)
    RPT = MC // NS
    BPT = B // NS
    assert BPT * NS == B and BPT % L == 0
    assert M - MC >= 0

    mesh = plsc.VectorSubcoreMesh(
        core_axis_name="c", subcore_axis_name="s",
        num_cores=NC, num_subcores=NS)

    @functools.partial(
        pl.kernel,
        out_type=jax.ShapeDtypeStruct((M, D), jnp.float32),
        mesh=mesh,
        compiler_params=pltpu.CompilerParams(needs_layout_passes=False),
        scratch_types=[
            pltpu.VMEM_SHARED((MC + NS, D), jnp.float32),
            pltpu.VMEM((BPT,), jnp.int32),
            pltpu.VMEM((BPT + L,), jnp.int32),
            pltpu.VMEM((BPT + L,), jnp.int32),
            pltpu.VMEM((L, D), jnp.float32),
            pltpu.VMEM((L,), jnp.float32),
        ],
    )
    def _ker(x_hbm, idx_hbm, src_hbm, alpha_hbm, out_hbm,
             acc, idx_v, selb_v, selr_v, gsrc_v, alpha_v):
        c = lax.axis_index("c")
        s = lax.axis_index("s")
        trash = MC + s

        pltpu.sync_copy(idx_hbm.at[pl.ds(pl.multiple_of(s * BPT, 8), BPT)], idx_v)
        pltpu.sync_copy(alpha_hbm, alpha_v)
        av = alpha_v[...]

        def chunk_body(i, _):
            chunk = NCH // NC * c + i
            lo = jnp.minimum(chunk * MC, M - MC)
            hi = lo + MC
            base = pl.multiple_of(lo + s * RPT, 8)
            sbase = pl.multiple_of(s * RPT, 8)

            pltpu.sync_copy(x_hbm.at[pl.ds(base, RPT)],
                            acc.at[pl.ds(sbase, RPT)])
            plsc.subcore_barrier()

            def sel_body(j, cnt):
                v = idx_v[pl.ds(j * L, L)]
                m = (v >= lo) & (v < hi)
                bsrc = s * BPT + j * L + lax.iota(jnp.int32, L)
                mi = jnp.where(m, jnp.int32(1), jnp.int32(0))
                pos = cnt + jnp.cumsum(mi) - 1
                plsc.store_scatter(selb_v, [pos], bsrc, mask=m)
                plsc.store_scatter(selr_v, [pos], v - lo, mask=m)
                return cnt + jnp.sum(mi)

            cnt = lax.fori_loop(0, BPT // L, sel_body, jnp.int32(0))

            selb_v[pl.ds(cnt, L)] = jnp.zeros((L,), jnp.int32)
            selr_v[pl.ds(cnt, L)] = jnp.broadcast_to(trash, (L,)).astype(jnp.int32)
            nb = (cnt + L - 1) // L

            def proc_body(j, _):
                bv = selb_v[pl.ds(j * L, L)]
                rv = selr_v[pl.ds(j * L, L)]
                pltpu.sync_copy(src_hbm.at[bv], gsrc_v)
                for r in range(L):
                    for cb in range(D // L):
                        sl = pl.ds(cb * L, L)
                        gsrc_v[r, sl] = gsrc_v[r, sl] * av
                pltpu.sync_copy(gsrc_v, acc.at[rv], add=True)
                return 0

            lax.fori_loop(0, nb, proc_body, 0)
            plsc.subcore_barrier()

            pltpu.sync_copy(acc.at[pl.ds(sbase, RPT)],
                            out_hbm.at[pl.ds(base, RPT)])
            plsc.subcore_barrier()
            return 0

        lax.fori_loop(0, NCH // NC, chunk_body, 0)

    return _ker


def kernel(x, dim, index, source, alpha, out):
    M, D = x.shape
    B = index.shape[0]
    alpha_arr = jnp.full((L,), alpha, jnp.float32)
    return _build(M, D, B)(x, index.astype(jnp.int32), source, alpha_arr)
